# Initial kernel scaffold; baseline (speedup 1.0000x reference)
import jax
import jax.numpy as jnp
from jax.experimental import pallas as pl


def kernel(h, edge_index, coord, edge_attr, W_e1, b_e1, W_e2, b_e2, W_n1, b_n1, W_n2, b_n2, W_c1, b_c1, W_c2):
    raise NotImplementedError("write your pallas kernel here")


# speedup vs baseline: 2.6058x; 2.6058x over previous
import functools

import jax
import jax.numpy as jnp
from jax import lax
from jax.experimental import pallas as pl
from jax.experimental.pallas import tpu as pltpu
from jax.experimental.pallas import tpu_sc as plsc

N = 10000
E = 320000
D = 128
H = 128
DE = 16
W = 144
NC = 2
NS = 16
NW = NC * NS
EW = E // NW
WIN = 128
NFULL = EW // WIN
TAIL = EW - NFULL * WIN
NPT = N // NS
ZR = 125


def _t1_body(h_ref, c_ref, w1r_ref, w1c_ref, be1_ref, p_ref, q_ref):
    h = h_ref[...]
    c = c_ref[...]
    zpad = jnp.zeros((h.shape[0], W - D - 3), jnp.float32)
    p = h @ w1r_ref[...] + be1_ref[...]
    q = h @ w1c_ref[...]
    p_ref[...] = jnp.concatenate([p, c, zpad], axis=1)
    q_ref[...] = jnp.concatenate([q, -c, zpad], axis=1)


def _t1(h, coord, W1r, W1c, b_e1):
    BN = 2000
    return pl.pallas_call(
        _t1_body,
        grid=(N // BN,),
        in_specs=[
            pl.BlockSpec((BN, D), lambda i: (i, 0)),
            pl.BlockSpec((BN, 3), lambda i: (i, 0)),
            pl.BlockSpec((D, H), lambda i: (0, 0)),
            pl.BlockSpec((D, H), lambda i: (0, 0)),
            pl.BlockSpec((1, H), lambda i: (0, 0)),
        ],
        out_specs=[
            pl.BlockSpec((BN, W), lambda i: (i, 0)),
            pl.BlockSpec((BN, W), lambda i: (i, 0)),
        ],
        out_shape=[
            jax.ShapeDtypeStruct((N, W), jnp.float32),
            jax.ShapeDtypeStruct((N, W), jnp.float32),
        ],
    )(h, coord, W1r, W1c, b_e1)


def _s1_body(row_hbm, col_hbm, p_hbm, q_hbm, hr_hbm, hc_hbm,
             ir, ic, gr, gc, irt, ict, grt, gct, sem):
    cid = lax.axis_index("c")
    sid = lax.axis_index("s")
    wbase = (sid * NC + cid) * EW

    def win(wi, _):
        base = wbase + wi * WIN
        pltpu.sync_copy(row_hbm.at[pl.ds(base, WIN)], ir)
        pltpu.sync_copy(col_hbm.at[pl.ds(base, WIN)], ic)
        pltpu.async_copy(p_hbm.at[ir], gr, sem).wait()
        pltpu.async_copy(q_hbm.at[ic], gc, sem).wait()
        pltpu.sync_copy(gr, hr_hbm.at[pl.ds(base, WIN)])
        pltpu.sync_copy(gc, hc_hbm.at[pl.ds(base, WIN)])
        return ()

    lax.fori_loop(0, NFULL, win, ())

    base = wbase + NFULL * WIN
    pltpu.sync_copy(row_hbm.at[pl.ds(base, TAIL)], irt)
    pltpu.sync_copy(col_hbm.at[pl.ds(base, TAIL)], ict)
    pltpu.async_copy(p_hbm.at[irt], grt, sem).wait()
    pltpu.async_copy(q_hbm.at[ict], gct, sem).wait()
    pltpu.sync_copy(grt, hr_hbm.at[pl.ds(base, TAIL)])
    pltpu.sync_copy(gct, hc_hbm.at[pl.ds(base, TAIL)])


def _s1(row, col, Pp, Qp):
    mesh = plsc.VectorSubcoreMesh(core_axis_name="c", subcore_axis_name="s")
    return pl.kernel(
        _s1_body,
        out_type=[
            jax.ShapeDtypeStruct((E, W), jnp.float32),
            jax.ShapeDtypeStruct((E, W), jnp.float32),
        ],
        mesh=mesh,
        compiler_params=pltpu.CompilerParams(use_tc_tiling_on_sc=False),
        scratch_types=[
            pltpu.VMEM((WIN,), jnp.int32),
            pltpu.VMEM((WIN,), jnp.int32),
            pltpu.VMEM((WIN, W), jnp.float32),
            pltpu.VMEM((WIN, W), jnp.float32),
            pltpu.VMEM((TAIL,), jnp.int32),
            pltpu.VMEM((TAIL,), jnp.int32),
            pltpu.VMEM((TAIL, W), jnp.float32),
            pltpu.VMEM((TAIL, W), jnp.float32),
            pltpu.SemaphoreType.DMA,
        ],
    )(row, col, Pp, Qp)


def _t2_body(hr_ref, hc_ref, ea_ref, w1a_ref, w1rad_ref, we2_ref, be2_ref,
             wc1_ref, bc1_ref, wc2_ref, f_ref):
    s = hr_ref[...] + hc_ref[...]
    diff = s[:, D:D + 3]
    radial = jnp.sum(diff * diff, axis=1, keepdims=True)
    m_in = s[:, :D] + radial * w1rad_ref[...] + ea_ref[...] @ w1a_ref[...]
    m = jax.nn.silu(m_in)
    ef = jax.nn.silu(m @ we2_ref[...] + be2_ref[...])
    c1 = jax.nn.silu(ef @ wc1_ref[...] + bc1_ref[...])
    phi = c1 @ wc2_ref[...]
    bt = ef.shape[0]
    extras = jnp.concatenate(
        [diff * phi, jnp.ones((bt, 1), jnp.float32),
         jnp.zeros((bt, W - D - 4), jnp.float32)], axis=1)
    f_ref[...] = jnp.concatenate([ef, extras], axis=1)


def _t2(HR, HC, edge_attr, W1a, w1rad, W_e2, b_e2, W_c1, b_c1, W_c2):
    BT = 1280
    return pl.pallas_call(
        _t2_body,
        grid=(E // BT,),
        in_specs=[
            pl.BlockSpec((BT, W), lambda i: (i, 0)),
            pl.BlockSpec((BT, W), lambda i: (i, 0)),
            pl.BlockSpec((BT, DE), lambda i: (i, 0)),
            pl.BlockSpec((DE, H), lambda i: (0, 0)),
            pl.BlockSpec((1, H), lambda i: (0, 0)),
            pl.BlockSpec((H, H), lambda i: (0, 0)),
            pl.BlockSpec((1, H), lambda i: (0, 0)),
            pl.BlockSpec((H, H), lambda i: (0, 0)),
            pl.BlockSpec((1, H), lambda i: (0, 0)),
            pl.BlockSpec((H, 1), lambda i: (0, 0)),
        ],
        out_specs=pl.BlockSpec((BT, W), lambda i: (i, 0)),
        out_shape=jax.ShapeDtypeStruct((E, W), jnp.float32),
    )(HR, HC, edge_attr, W1a, w1rad, W_e2, b_e2, W_c1, b_c1, W_c2)


def _s2_body(row_hbm, f_hbm, acc2_hbm, ib, fb, ibt, fbt, zb, acc, sem):
    cid = lax.axis_index("c")
    sid = lax.axis_index("s")
    wbase = (sid * NC + cid) * EW

    zero16 = jnp.zeros((16,), jnp.float32)

    def zrow(r, _):
        for cc in range(W // 16):
            zb[r, pl.ds(cc * 16, 16)] = zero16
        return ()

    lax.fori_loop(0, ZR, zrow, ())

    def zchunk(k, _):
        pltpu.sync_copy(zb, acc.at[pl.ds(sid * NPT + k * ZR, ZR)])
        return ()

    lax.fori_loop(0, NPT // ZR, zchunk, ())
    plsc.subcore_barrier()

    def win(wi, _):
        base = wbase + wi * WIN
        pltpu.sync_copy(row_hbm.at[pl.ds(base, WIN)], ib)
        pltpu.sync_copy(f_hbm.at[pl.ds(base, WIN)], fb)
        pltpu.sync_copy(fb, acc.at[ib], add=True)
        return ()

    lax.fori_loop(0, NFULL, win, ())

    base = wbase + NFULL * WIN
    pltpu.sync_copy(row_hbm.at[pl.ds(base, TAIL)], ibt)
    pltpu.sync_copy(f_hbm.at[pl.ds(base, TAIL)], fbt)
    pltpu.sync_copy(fbt, acc.at[ibt], add=True)

    plsc.subcore_barrier()

    def flush(k, _):
        off = sid * NPT + k * ZR
        pltpu.sync_copy(acc.at[pl.ds(off, ZR)], acc2_hbm.at[cid, pl.ds(off, ZR)])
        return ()

    lax.fori_loop(0, NPT // ZR, flush, ())


def _s2(row, F):
    mesh = plsc.VectorSubcoreMesh(core_axis_name="c", subcore_axis_name="s")
    return pl.kernel(
        _s2_body,
        out_type=jax.ShapeDtypeStruct((NC, N, W), jnp.float32),
        mesh=mesh,
        compiler_params=pltpu.CompilerParams(use_tc_tiling_on_sc=False),
        scratch_types=[
            pltpu.VMEM((WIN,), jnp.int32),
            pltpu.VMEM((WIN, W), jnp.float32),
            pltpu.VMEM((TAIL,), jnp.int32),
            pltpu.VMEM((TAIL, W), jnp.float32),
            pltpu.VMEM((ZR, W), jnp.float32),
            pltpu.VMEM_SHARED((N, W), jnp.float32),
            pltpu.SemaphoreType.DMA,
        ],
    )(row, F)


def _t3_body(acc2_ref, h_ref, c_ref, wn1h_ref, wn1a_ref, bn1_ref, wn2_ref,
             bn2_ref, ho_ref, co_ref):
    acc = acc2_ref[0] + acc2_ref[1]
    agg_h = acc[:, :D]
    sums = acc[:, D:D + 3]
    cnt = acc[:, D + 3:D + 4]
    h = h_ref[...]
    t = jax.nn.silu(h @ wn1h_ref[...] + agg_h @ wn1a_ref[...] + bn1_ref[...])
    ho_ref[...] = h + t @ wn2_ref[...] + bn2_ref[...]
    co_ref[...] = c_ref[...] + sums / jnp.maximum(cnt, 1.0)


def _t3(ACC2, h, coord, Wn1h, Wn1a, b_n1, W_n2, b_n2):
    BN = 2000
    return pl.pallas_call(
        _t3_body,
        grid=(N // BN,),
        in_specs=[
            pl.BlockSpec((NC, BN, W), lambda i: (0, i, 0)),
            pl.BlockSpec((BN, D), lambda i: (i, 0)),
            pl.BlockSpec((BN, 3), lambda i: (i, 0)),
            pl.BlockSpec((D, H), lambda i: (0, 0)),
            pl.BlockSpec((H, H), lambda i: (0, 0)),
            pl.BlockSpec((1, H), lambda i: (0, 0)),
            pl.BlockSpec((H, D), lambda i: (0, 0)),
            pl.BlockSpec((1, D), lambda i: (0, 0)),
        ],
        out_specs=[
            pl.BlockSpec((BN, D), lambda i: (i, 0)),
            pl.BlockSpec((BN, 3), lambda i: (i, 0)),
        ],
        out_shape=[
            jax.ShapeDtypeStruct((N, D), jnp.float32),
            jax.ShapeDtypeStruct((N, 3), jnp.float32),
        ],
    )(ACC2, h, coord, Wn1h, Wn1a, b_n1, W_n2, b_n2)


@jax.jit
def kernel(h, edge_index, coord, edge_attr, W_e1, b_e1, W_e2, b_e2,
           W_n1, b_n1, W_n2, b_n2, W_c1, b_c1, W_c2):
    row = edge_index[0]
    col = edge_index[1]

    W1r = W_e1[:D]
    W1c = W_e1[D:2 * D]
    w1rad = W_e1[2 * D:2 * D + 1]
    W1a = W_e1[2 * D + 1:]

    Pp, Qp = _t1(h, coord, W1r, W1c, b_e1.reshape(1, H))
    HR, HC = _s1(row, col, Pp, Qp)
    F = _t2(HR, HC, edge_attr, W1a, w1rad, W_e2, b_e2.reshape(1, H),
            W_c1, b_c1.reshape(1, H), W_c2)
    ACC2 = _s2(row, F)
    h_out, coord_out = _t3(ACC2, h, coord, W_n1[:D], W_n1[D:],
                           b_n1.reshape(1, H), W_n2, b_n2.reshape(1, D))
    return (h_out, coord_out, edge_attr)


# speedup vs baseline: 4.0249x; 1.5446x over previous
import functools

import jax
import jax.numpy as jnp
from jax import lax
from jax.experimental import pallas as pl
from jax.experimental.pallas import tpu as pltpu
from jax.experimental.pallas import tpu_sc as plsc

N = 10000
E = 320000
D = 128
H = 128
DE = 16
X = 16
NC = 2
NS = 16
NW = NC * NS
EW = E // NW
WIN = 128
NFULL = EW // WIN
TAIL = EW - NFULL * WIN
NPT = N // NS
ZR = 125

_SC_PARAMS = pltpu.CompilerParams(use_tc_tiling_on_sc=False)


def _t1_body(h_ref, w1r_ref, w1c_ref, be1_ref, p_ref, q_ref):
    h = h_ref[...]
    p_ref[...] = h @ w1r_ref[...] + be1_ref[...]
    q_ref[...] = h @ w1c_ref[...]


def _t1(h, W1r, W1c, b_e1):
    BN = 2000
    return pl.pallas_call(
        _t1_body,
        grid=(N // BN,),
        in_specs=[
            pl.BlockSpec((BN, D), lambda i: (i, 0)),
            pl.BlockSpec((D, H), lambda i: (0, 0)),
            pl.BlockSpec((D, H), lambda i: (0, 0)),
            pl.BlockSpec((1, H), lambda i: (0, 0)),
        ],
        out_specs=[
            pl.BlockSpec((BN, H), lambda i: (i, 0)),
            pl.BlockSpec((BN, H), lambda i: (i, 0)),
        ],
        out_shape=[
            jax.ShapeDtypeStruct((N, H), jnp.float32),
            jax.ShapeDtypeStruct((N, H), jnp.float32),
        ],
    )(h, W1r, W1c, b_e1)


def _s1_body(row_hbm, col_hbm, p_hbm, q_hbm, cx_hbm, hr_hbm, hc_hbm, df_hbm,
             ir, ic, gr, gc, cr, cc, sem):
    cid = lax.axis_index("c")
    sid = lax.axis_index("s")
    wbase = (sid * NC + cid) * EW

    def do_win(base, size):
        pltpu.sync_copy(row_hbm.at[pl.ds(base, size)], ir.at[pl.ds(0, size)])
        pltpu.sync_copy(col_hbm.at[pl.ds(base, size)], ic.at[pl.ds(0, size)])
        irs = ir.at[pl.ds(0, size)] if size != WIN else ir
        ics = ic.at[pl.ds(0, size)] if size != WIN else ic
        cp1 = pltpu.async_copy(p_hbm.at[irs], gr.at[pl.ds(0, size)], sem)
        cp2 = pltpu.async_copy(q_hbm.at[ics], gc.at[pl.ds(0, size)], sem)
        cp3 = pltpu.async_copy(cx_hbm.at[irs], cr.at[pl.ds(0, size)], sem)
        cp4 = pltpu.async_copy(cx_hbm.at[ics], cc.at[pl.ds(0, size)], sem)
        cp1.wait()
        cp2.wait()
        cp3.wait()
        cp4.wait()

        def drow(r, _):
            cr[r, :] = cr[r, :] - cc[r, :]
            return ()

        lax.fori_loop(0, size, drow, ())
        pltpu.sync_copy(gr.at[pl.ds(0, size)], hr_hbm.at[pl.ds(base, size)])
        pltpu.sync_copy(gc.at[pl.ds(0, size)], hc_hbm.at[pl.ds(base, size)])
        pltpu.sync_copy(cr.at[pl.ds(0, size)], df_hbm.at[pl.ds(base, size)])

    def win(wi, _):
        do_win(wbase + wi * WIN, WIN)
        return ()

    lax.fori_loop(0, NFULL, win, ())
    do_win(wbase + NFULL * WIN, TAIL)


def _s1(row, col, P, Q, CX):
    mesh = plsc.VectorSubcoreMesh(core_axis_name="c", subcore_axis_name="s")
    return pl.kernel(
        _s1_body,
        out_type=[
            jax.ShapeDtypeStruct((E, H), jnp.float32),
            jax.ShapeDtypeStruct((E, H), jnp.float32),
            jax.ShapeDtypeStruct((E, X), jnp.float32),
        ],
        mesh=mesh,
        compiler_params=_SC_PARAMS,
        scratch_types=[
            pltpu.VMEM((WIN,), jnp.int32),
            pltpu.VMEM((WIN,), jnp.int32),
            pltpu.VMEM((WIN, H), jnp.float32),
            pltpu.VMEM((WIN, H), jnp.float32),
            pltpu.VMEM((WIN, X), jnp.float32),
            pltpu.VMEM((WIN, X), jnp.float32),
            pltpu.SemaphoreType.DMA,
        ],
    )(row, col, P, Q, CX)


def _t2_body(hr_ref, hc_ref, df_ref, ea_ref, w1a_ref, w1rad_ref, we2_ref,
             be2_ref, wc1_ref, bc1_ref, wc2_ref, f_ref, fx_ref):
    diff = df_ref[:, :3]
    radial = jnp.sum(diff * diff, axis=1, keepdims=True)
    m_in = (hr_ref[...] + hc_ref[...]) + radial * w1rad_ref[...] \
        + ea_ref[...] @ w1a_ref[...]
    m = jax.nn.silu(m_in)
    ef = jax.nn.silu(m @ we2_ref[...] + be2_ref[...])
    c1 = jax.nn.silu(ef @ wc1_ref[...] + bc1_ref[...])
    phi = c1 @ wc2_ref[...]
    bt = ef.shape[0]
    f_ref[...] = ef
    fx_ref[...] = jnp.concatenate(
        [diff * phi, jnp.ones((bt, 1), jnp.float32),
         jnp.zeros((bt, X - 4), jnp.float32)], axis=1)


def _t2(HR, HC, DF, edge_attr, W1a, w1rad, W_e2, b_e2, W_c1, b_c1, W_c2):
    BT = 1280
    return pl.pallas_call(
        _t2_body,
        grid=(E // BT,),
        in_specs=[
            pl.BlockSpec((BT, H), lambda i: (i, 0)),
            pl.BlockSpec((BT, H), lambda i: (i, 0)),
            pl.BlockSpec((BT, X), lambda i: (i, 0)),
            pl.BlockSpec((BT, DE), lambda i: (i, 0)),
            pl.BlockSpec((DE, H), lambda i: (0, 0)),
            pl.BlockSpec((1, H), lambda i: (0, 0)),
            pl.BlockSpec((H, H), lambda i: (0, 0)),
            pl.BlockSpec((1, H), lambda i: (0, 0)),
            pl.BlockSpec((H, H), lambda i: (0, 0)),
            pl.BlockSpec((1, H), lambda i: (0, 0)),
            pl.BlockSpec((H, 1), lambda i: (0, 0)),
        ],
        out_specs=[
            pl.BlockSpec((BT, H), lambda i: (i, 0)),
            pl.BlockSpec((BT, X), lambda i: (i, 0)),
        ],
        out_shape=[
            jax.ShapeDtypeStruct((E, H), jnp.float32),
            jax.ShapeDtypeStruct((E, X), jnp.float32),
        ],
    )(HR, HC, DF, edge_attr, W1a, w1rad, W_e2, b_e2, W_c1, b_c1, W_c2)


def _s2_body(row_hbm, f_hbm, fx_hbm, acc2_hbm, accx2_hbm,
             ib, fb, fxb, zb, zxb, acc, accx, sem):
    cid = lax.axis_index("c")
    sid = lax.axis_index("s")
    wbase = (sid * NC + cid) * EW

    zero16 = jnp.zeros((16,), jnp.float32)

    def zrow(r, _):
        for ch in range(H // 16):
            zb[r, pl.ds(ch * 16, 16)] = zero16
        zxb[r, :] = zero16
        return ()

    lax.fori_loop(0, ZR, zrow, ())

    def zchunk(k, _):
        pltpu.sync_copy(zb, acc.at[pl.ds(sid * NPT + k * ZR, ZR)])
        pltpu.sync_copy(zxb, accx.at[pl.ds(sid * NPT + k * ZR, ZR)])
        return ()

    lax.fori_loop(0, NPT // ZR, zchunk, ())
    plsc.subcore_barrier()

    def do_win(base, size):
        pltpu.sync_copy(row_hbm.at[pl.ds(base, size)], ib.at[pl.ds(0, size)])
        ibs = ib.at[pl.ds(0, size)] if size != WIN else ib
        pltpu.sync_copy(f_hbm.at[pl.ds(base, size)], fb.at[pl.ds(0, size)])
        pltpu.sync_copy(fx_hbm.at[pl.ds(base, size)], fxb.at[pl.ds(0, size)])
        pltpu.sync_copy(fb.at[pl.ds(0, size)], acc.at[ibs], add=True)
        pltpu.sync_copy(fxb.at[pl.ds(0, size)], accx.at[ibs], add=True)

    def win(wi, _):
        do_win(wbase + wi * WIN, WIN)
        return ()

    lax.fori_loop(0, NFULL, win, ())
    do_win(wbase + NFULL * WIN, TAIL)

    plsc.subcore_barrier()

    def flush(k, _):
        off = sid * NPT + k * ZR
        pltpu.sync_copy(acc.at[pl.ds(off, ZR)], acc2_hbm.at[cid, pl.ds(off, ZR)])
        pltpu.sync_copy(accx.at[pl.ds(off, ZR)],
                        accx2_hbm.at[cid, pl.ds(off, ZR)])
        return ()

    lax.fori_loop(0, NPT // ZR, flush, ())


def _s2(row, F, FX):
    mesh = plsc.VectorSubcoreMesh(core_axis_name="c", subcore_axis_name="s")
    return pl.kernel(
        _s2_body,
        out_type=[
            jax.ShapeDtypeStruct((NC, N, H), jnp.float32),
            jax.ShapeDtypeStruct((NC, N, X), jnp.float32),
        ],
        mesh=mesh,
        compiler_params=_SC_PARAMS,
        scratch_types=[
            pltpu.VMEM((WIN,), jnp.int32),
            pltpu.VMEM((WIN, H), jnp.float32),
            pltpu.VMEM((WIN, X), jnp.float32),
            pltpu.VMEM((ZR, H), jnp.float32),
            pltpu.VMEM((ZR, X), jnp.float32),
            pltpu.VMEM_SHARED((N, H), jnp.float32),
            pltpu.VMEM_SHARED((N, X), jnp.float32),
            pltpu.SemaphoreType.DMA,
        ],
    )(row, F, FX)


def _t3_body(acc2_ref, accx2_ref, h_ref, c_ref, wn1h_ref, wn1a_ref, bn1_ref,
             wn2_ref, bn2_ref, ho_ref, co_ref):
    agg_h = acc2_ref[0] + acc2_ref[1]
    accx = accx2_ref[0] + accx2_ref[1]
    sums = accx[:, :3]
    cnt = accx[:, 3:4]
    h = h_ref[...]
    t = jax.nn.silu(h @ wn1h_ref[...] + agg_h @ wn1a_ref[...] + bn1_ref[...])
    ho_ref[...] = h + t @ wn2_ref[...] + bn2_ref[...]
    co_ref[...] = c_ref[...] + sums / jnp.maximum(cnt, 1.0)


def _t3(ACC2, ACCX2, h, coord, Wn1h, Wn1a, b_n1, W_n2, b_n2):
    BN = 2000
    return pl.pallas_call(
        _t3_body,
        grid=(N // BN,),
        in_specs=[
            pl.BlockSpec((NC, BN, H), lambda i: (0, i, 0)),
            pl.BlockSpec((NC, BN, X), lambda i: (0, i, 0)),
            pl.BlockSpec((BN, D), lambda i: (i, 0)),
            pl.BlockSpec((BN, 3), lambda i: (i, 0)),
            pl.BlockSpec((D, H), lambda i: (0, 0)),
            pl.BlockSpec((H, H), lambda i: (0, 0)),
            pl.BlockSpec((1, H), lambda i: (0, 0)),
            pl.BlockSpec((H, D), lambda i: (0, 0)),
            pl.BlockSpec((1, D), lambda i: (0, 0)),
        ],
        out_specs=[
            pl.BlockSpec((BN, D), lambda i: (i, 0)),
            pl.BlockSpec((BN, 3), lambda i: (i, 0)),
        ],
        out_shape=[
            jax.ShapeDtypeStruct((N, D), jnp.float32),
            jax.ShapeDtypeStruct((N, 3), jnp.float32),
        ],
    )(ACC2, ACCX2, h, coord, Wn1h, Wn1a, b_n1, W_n2, b_n2)


@jax.jit
def kernel(h, edge_index, coord, edge_attr, W_e1, b_e1, W_e2, b_e2,
           W_n1, b_n1, W_n2, b_n2, W_c1, b_c1, W_c2):
    row = edge_index[0]
    col = edge_index[1]

    W1r = W_e1[:D]
    W1c = W_e1[D:2 * D]
    w1rad = W_e1[2 * D:2 * D + 1]
    W1a = W_e1[2 * D + 1:]

    CX = jnp.pad(coord, ((0, 0), (0, X - 3)))

    P, Q = _t1(h, W1r, W1c, b_e1.reshape(1, H))
    HR, HC, DF = _s1(row, col, P, Q, CX)
    F, FX = _t2(HR, HC, DF, edge_attr, W1a, w1rad, W_e2, b_e2.reshape(1, H),
                W_c1, b_c1.reshape(1, H), W_c2)
    ACC2, ACCX2 = _s2(row, F, FX)
    h_out, coord_out = _t3(ACC2, ACCX2, h, coord, W_n1[:D], W_n1[D:],
                           b_n1.reshape(1, H), W_n2, b_n2.reshape(1, D))
    return (h_out, coord_out, edge_attr)


# speedup vs baseline: 4.1434x; 1.0294x over previous
import functools

import jax
import jax.numpy as jnp
from jax import lax
from jax.experimental import pallas as pl
from jax.experimental.pallas import tpu as pltpu
from jax.experimental.pallas import tpu_sc as plsc

N = 10000
E = 320000
D = 128
H = 128
DE = 16
X = 16
NC = 2
NS = 16
NW = NC * NS
EW = E // NW
WIN = 128
NFULL = EW // WIN
TAIL = EW - NFULL * WIN
NPT = N // NS
ZR = 125

_SC_PARAMS = pltpu.CompilerParams(use_tc_tiling_on_sc=False)


def _t1_body(h_ref, w1r_ref, w1c_ref, be1_ref, p_ref, q_ref):
    h = h_ref[...]
    p_ref[...] = h @ w1r_ref[...] + be1_ref[...]
    q_ref[...] = h @ w1c_ref[...]


def _t1(h, W1r, W1c, b_e1):
    BN = 2000
    return pl.pallas_call(
        _t1_body,
        grid=(N // BN,),
        in_specs=[
            pl.BlockSpec((BN, D), lambda i: (i, 0)),
            pl.BlockSpec((D, H), lambda i: (0, 0)),
            pl.BlockSpec((D, H), lambda i: (0, 0)),
            pl.BlockSpec((1, H), lambda i: (0, 0)),
        ],
        out_specs=[
            pl.BlockSpec((BN, H), lambda i: (i, 0)),
            pl.BlockSpec((BN, H), lambda i: (i, 0)),
        ],
        out_shape=[
            jax.ShapeDtypeStruct((N, H), jnp.float32),
            jax.ShapeDtypeStruct((N, H), jnp.float32),
        ],
    )(h, W1r, W1c, b_e1)


def _s1_body(row_hbm, col_hbm, p_hbm, q_hbm, cx_hbm, g_hbm, df_hbm,
             ir, ic, gr, gc, cr, cc, sem):
    cid = lax.axis_index("c")
    sid = lax.axis_index("s")
    wbase = (sid * NC + cid) * EW

    def do_win(base, size):
        pltpu.sync_copy(row_hbm.at[pl.ds(base, size)], ir.at[pl.ds(0, size)])
        pltpu.sync_copy(col_hbm.at[pl.ds(base, size)], ic.at[pl.ds(0, size)])
        irs = ir.at[pl.ds(0, size)] if size != WIN else ir
        ics = ic.at[pl.ds(0, size)] if size != WIN else ic
        cp1 = pltpu.async_copy(p_hbm.at[irs], gr.at[pl.ds(0, size)], sem)
        cp2 = pltpu.async_copy(q_hbm.at[ics], gc.at[pl.ds(0, size)], sem)
        cp3 = pltpu.async_copy(cx_hbm.at[irs], cr.at[pl.ds(0, size)], sem)
        cp4 = pltpu.async_copy(cx_hbm.at[ics], cc.at[pl.ds(0, size)], sem)
        cp1.wait()
        cp2.wait()
        cp3.wait()
        cp4.wait()

        @plsc.parallel_loop(0, size, unroll=8)
        def _(r):
            for ch in range(H // 16):
                sl = pl.ds(ch * 16, 16)
                gr[r, sl] = gr[r, sl] + gc[r, sl]
            cr[r, :] = cr[r, :] - cc[r, :]

        pltpu.sync_copy(gr.at[pl.ds(0, size)], g_hbm.at[pl.ds(base, size)])
        pltpu.sync_copy(cr.at[pl.ds(0, size)], df_hbm.at[pl.ds(base, size)])

    def win(wi, _):
        do_win(wbase + wi * WIN, WIN)
        return ()

    lax.fori_loop(0, NFULL, win, ())
    do_win(wbase + NFULL * WIN, TAIL)


def _s1(row, col, P, Q, CX):
    mesh = plsc.VectorSubcoreMesh(core_axis_name="c", subcore_axis_name="s")
    return pl.kernel(
        _s1_body,
        out_type=[
            jax.ShapeDtypeStruct((E, H), jnp.float32),
            jax.ShapeDtypeStruct((E, X), jnp.float32),
        ],
        mesh=mesh,
        compiler_params=_SC_PARAMS,
        scratch_types=[
            pltpu.VMEM((WIN,), jnp.int32),
            pltpu.VMEM((WIN,), jnp.int32),
            pltpu.VMEM((WIN, H), jnp.float32),
            pltpu.VMEM((WIN, H), jnp.float32),
            pltpu.VMEM((WIN, X), jnp.float32),
            pltpu.VMEM((WIN, X), jnp.float32),
            pltpu.SemaphoreType.DMA,
        ],
    )(row, col, P, Q, CX)


def _t2_body(g_ref, df_ref, ea_ref, w1a_ref, w1rad_ref, we2_ref,
             be2_ref, wc1_ref, bc1_ref, wc2_ref, f_ref, fx_ref):
    diff = df_ref[:, :3]
    radial = jnp.sum(diff * diff, axis=1, keepdims=True)
    m_in = g_ref[...] + radial * w1rad_ref[...] + ea_ref[...] @ w1a_ref[...]
    m = jax.nn.silu(m_in)
    ef = jax.nn.silu(m @ we2_ref[...] + be2_ref[...])
    c1 = jax.nn.silu(ef @ wc1_ref[...] + bc1_ref[...])
    phi = c1 @ wc2_ref[...]
    bt = ef.shape[0]
    f_ref[...] = ef
    fx_ref[...] = jnp.concatenate(
        [diff * phi, jnp.ones((bt, 1), jnp.float32),
         jnp.zeros((bt, X - 4), jnp.float32)], axis=1)


def _t2(G, DF, edge_attr, W1a, w1rad, W_e2, b_e2, W_c1, b_c1, W_c2):
    BT = 1280
    return pl.pallas_call(
        _t2_body,
        grid=(E // BT,),
        in_specs=[
            pl.BlockSpec((BT, H), lambda i: (i, 0)),
            pl.BlockSpec((BT, X), lambda i: (i, 0)),
            pl.BlockSpec((BT, DE), lambda i: (i, 0)),
            pl.BlockSpec((DE, H), lambda i: (0, 0)),
            pl.BlockSpec((1, H), lambda i: (0, 0)),
            pl.BlockSpec((H, H), lambda i: (0, 0)),
            pl.BlockSpec((1, H), lambda i: (0, 0)),
            pl.BlockSpec((H, H), lambda i: (0, 0)),
            pl.BlockSpec((1, H), lambda i: (0, 0)),
            pl.BlockSpec((H, 1), lambda i: (0, 0)),
        ],
        out_specs=[
            pl.BlockSpec((BT, H), lambda i: (i, 0)),
            pl.BlockSpec((BT, X), lambda i: (i, 0)),
        ],
        out_shape=[
            jax.ShapeDtypeStruct((E, H), jnp.float32),
            jax.ShapeDtypeStruct((E, X), jnp.float32),
        ],
    )(G, DF, edge_attr, W1a, w1rad, W_e2, b_e2, W_c1, b_c1, W_c2)


def _s2_body(row_hbm, f_hbm, fx_hbm, acc2_hbm, accx2_hbm,
             ib, fb, fxb, zb, zxb, acc, accx, sem):
    cid = lax.axis_index("c")
    sid = lax.axis_index("s")
    wbase = (sid * NC + cid) * EW

    zero16 = jnp.zeros((16,), jnp.float32)

    def zrow(r, _):
        for ch in range(H // 16):
            zb[r, pl.ds(ch * 16, 16)] = zero16
        zxb[r, :] = zero16
        return ()

    lax.fori_loop(0, ZR, zrow, ())

    def zchunk(k, _):
        pltpu.sync_copy(zb, acc.at[pl.ds(sid * NPT + k * ZR, ZR)])
        pltpu.sync_copy(zxb, accx.at[pl.ds(sid * NPT + k * ZR, ZR)])
        return ()

    lax.fori_loop(0, NPT // ZR, zchunk, ())
    plsc.subcore_barrier()

    def do_win(base, size):
        pltpu.sync_copy(row_hbm.at[pl.ds(base, size)], ib.at[pl.ds(0, size)])
        ibs = ib.at[pl.ds(0, size)] if size != WIN else ib
        pltpu.sync_copy(f_hbm.at[pl.ds(base, size)], fb.at[pl.ds(0, size)])
        pltpu.sync_copy(fx_hbm.at[pl.ds(base, size)], fxb.at[pl.ds(0, size)])
        pltpu.sync_copy(fb.at[pl.ds(0, size)], acc.at[ibs], add=True)
        pltpu.sync_copy(fxb.at[pl.ds(0, size)], accx.at[ibs], add=True)

    def win(wi, _):
        do_win(wbase + wi * WIN, WIN)
        return ()

    lax.fori_loop(0, NFULL, win, ())
    do_win(wbase + NFULL * WIN, TAIL)

    plsc.subcore_barrier()

    def flush(k, _):
        off = sid * NPT + k * ZR
        pltpu.sync_copy(acc.at[pl.ds(off, ZR)], acc2_hbm.at[cid, pl.ds(off, ZR)])
        pltpu.sync_copy(accx.at[pl.ds(off, ZR)],
                        accx2_hbm.at[cid, pl.ds(off, ZR)])
        return ()

    lax.fori_loop(0, NPT // ZR, flush, ())


def _s2(row, F, FX):
    mesh = plsc.VectorSubcoreMesh(core_axis_name="c", subcore_axis_name="s")
    return pl.kernel(
        _s2_body,
        out_type=[
            jax.ShapeDtypeStruct((NC, N, H), jnp.float32),
            jax.ShapeDtypeStruct((NC, N, X), jnp.float32),
        ],
        mesh=mesh,
        compiler_params=_SC_PARAMS,
        scratch_types=[
            pltpu.VMEM((WIN,), jnp.int32),
            pltpu.VMEM((WIN, H), jnp.float32),
            pltpu.VMEM((WIN, X), jnp.float32),
            pltpu.VMEM((ZR, H), jnp.float32),
            pltpu.VMEM((ZR, X), jnp.float32),
            pltpu.VMEM_SHARED((N, H), jnp.float32),
            pltpu.VMEM_SHARED((N, X), jnp.float32),
            pltpu.SemaphoreType.DMA,
        ],
    )(row, F, FX)


def _t3_body(acc2_ref, accx2_ref, h_ref, c_ref, wn1h_ref, wn1a_ref, bn1_ref,
             wn2_ref, bn2_ref, ho_ref, co_ref):
    agg_h = acc2_ref[0] + acc2_ref[1]
    accx = accx2_ref[0] + accx2_ref[1]
    sums = accx[:, :3]
    cnt = accx[:, 3:4]
    h = h_ref[...]
    t = jax.nn.silu(h @ wn1h_ref[...] + agg_h @ wn1a_ref[...] + bn1_ref[...])
    ho_ref[...] = h + t @ wn2_ref[...] + bn2_ref[...]
    co_ref[...] = c_ref[...] + sums / jnp.maximum(cnt, 1.0)


def _t3(ACC2, ACCX2, h, coord, Wn1h, Wn1a, b_n1, W_n2, b_n2):
    BN = 2000
    return pl.pallas_call(
        _t3_body,
        grid=(N // BN,),
        in_specs=[
            pl.BlockSpec((NC, BN, H), lambda i: (0, i, 0)),
            pl.BlockSpec((NC, BN, X), lambda i: (0, i, 0)),
            pl.BlockSpec((BN, D), lambda i: (i, 0)),
            pl.BlockSpec((BN, 3), lambda i: (i, 0)),
            pl.BlockSpec((D, H), lambda i: (0, 0)),
            pl.BlockSpec((H, H), lambda i: (0, 0)),
            pl.BlockSpec((1, H), lambda i: (0, 0)),
            pl.BlockSpec((H, D), lambda i: (0, 0)),
            pl.BlockSpec((1, D), lambda i: (0, 0)),
        ],
        out_specs=[
            pl.BlockSpec((BN, D), lambda i: (i, 0)),
            pl.BlockSpec((BN, 3), lambda i: (i, 0)),
        ],
        out_shape=[
            jax.ShapeDtypeStruct((N, D), jnp.float32),
            jax.ShapeDtypeStruct((N, 3), jnp.float32),
        ],
    )(ACC2, ACCX2, h, coord, Wn1h, Wn1a, b_n1, W_n2, b_n2)


@jax.jit
def kernel(h, edge_index, coord, edge_attr, W_e1, b_e1, W_e2, b_e2,
           W_n1, b_n1, W_n2, b_n2, W_c1, b_c1, W_c2):
    row = edge_index[0]
    col = edge_index[1]

    W1r = W_e1[:D]
    W1c = W_e1[D:2 * D]
    w1rad = W_e1[2 * D:2 * D + 1]
    W1a = W_e1[2 * D + 1:]

    CX = jnp.pad(coord, ((0, 0), (0, X - 3)))

    P, Q = _t1(h, W1r, W1c, b_e1.reshape(1, H))
    G, DF = _s1(row, col, P, Q, CX)
    F, FX = _t2(G, DF, edge_attr, W1a, w1rad, W_e2, b_e2.reshape(1, H),
                W_c1, b_c1.reshape(1, H), W_c2)
    ACC2, ACCX2 = _s2(row, F, FX)
    h_out, coord_out = _t3(ACC2, ACCX2, h, coord, W_n1[:D], W_n1[D:],
                           b_n1.reshape(1, H), W_n2, b_n2.reshape(1, D))
    return (h_out, coord_out, edge_attr)
